# Initial kernel scaffold; baseline (speedup 1.0000x reference)
import jax
import jax.numpy as jnp
from jax.experimental import pallas as pl


def kernel(indices, per_sample_weights, weight):
    raise NotImplementedError("write your pallas kernel here")


# speedup vs baseline: 15.6974x; 15.6974x over previous
import functools

import jax
import jax.numpy as jnp
from jax.experimental import pallas as pl

B = 16384
L = 50
VOCAB = 256
DIM = 256
BLK = 1024


def _body(idx_ref, psw_ref, w_ref, out_ref):
    idx = idx_ref[...]
    psw = psw_ref[...]
    vocab_iota = jax.lax.broadcasted_iota(jnp.int32, (BLK, VOCAB), 1)
    coeff = jnp.zeros((BLK, VOCAB), jnp.float32)
    for l in range(L):
        hit = idx[:, l : l + 1] == vocab_iota
        coeff = coeff + jnp.where(hit, psw[:, l : l + 1], 0.0)
    out_ref[...] = jnp.dot(coeff, w_ref[...], preferred_element_type=jnp.float32)


@jax.jit
def kernel(indices, per_sample_weights, weight):
    grid = (B // BLK,)
    return pl.pallas_call(
        _body,
        grid=grid,
        in_specs=[
            pl.BlockSpec((BLK, L), lambda i: (i, 0)),
            pl.BlockSpec((BLK, L), lambda i: (i, 0)),
            pl.BlockSpec((VOCAB, DIM), lambda i: (0, 0)),
        ],
        out_specs=pl.BlockSpec((BLK, DIM), lambda i: (i, 0)),
        out_shape=jax.ShapeDtypeStruct((B, DIM), jnp.float32),
    )(indices.astype(jnp.int32), per_sample_weights, weight)


# speedup vs baseline: 22.5726x; 1.4380x over previous
import functools

import jax
import jax.numpy as jnp
from jax import lax
from jax.experimental import pallas as pl
from jax.experimental.pallas import tpu as pltpu
from jax.experimental.pallas import tpu_sc as plsc

B = 16384
L = 50
VOCAB = 256
DIM = 256
LP = 64

_NC, _NS = 2, 16
_NW = _NC * _NS
_RPW = B // _NW
_SUB = 128
_NSUB = _RPW // _SUB
_GRP = _SUB // 16


def _sc_body(idx_hbm, psw_hbm, coeff_hbm, idx_v, psw_v, acc_v):
    wid = lax.axis_index("s") * _NC + lax.axis_index("c")
    base = wid * _RPW
    lane = lax.iota(jnp.int32, 16)
    gath_base = lane * LP
    row_base = lane * VOCAB
    zeros16 = jnp.zeros((16,), jnp.float32)

    for sub in range(_NSUB):
        r0 = base + sub * _SUB
        pltpu.sync_copy(idx_hbm.at[pl.ds(r0 * LP, _SUB * LP)], idx_v)
        pltpu.sync_copy(psw_hbm.at[pl.ds(r0 * LP, _SUB * LP)], psw_v)

        def zbody(i, c):
            acc_v[pl.ds(i * 16, 16)] = zeros16
            return c

        lax.fori_loop(0, _SUB * VOCAB // 16, zbody, 0)

        def lbody(l, c):
            for g in range(_GRP):
                gidx = gath_base + (g * 16 * LP + l)
                ivals = plsc.load_gather(idx_v, [gidx])
                pvals = plsc.load_gather(psw_v, [gidx])
                dest = (row_base + g * 16 * VOCAB) + ivals
                plsc.addupdate_scatter(acc_v, [dest], pvals)
            return c

        lax.fori_loop(0, LP, lbody, 0)
        pltpu.sync_copy(acc_v, coeff_hbm.at[pl.ds(r0 * VOCAB, _SUB * VOCAB)])


_sc_coeff = functools.partial(
    pl.kernel,
    out_type=jax.ShapeDtypeStruct((B * VOCAB,), jnp.float32),
    mesh=plsc.VectorSubcoreMesh(
        core_axis_name="c", subcore_axis_name="s", num_cores=_NC, num_subcores=_NS
    ),
    scratch_types=[
        pltpu.VMEM((_SUB * LP,), jnp.int32),
        pltpu.VMEM((_SUB * LP,), jnp.float32),
        pltpu.VMEM((_SUB * VOCAB,), jnp.float32),
    ],
    compiler_params=pltpu.CompilerParams(needs_layout_passes=False),
)(_sc_body)


_MBLK = 2048


def _mm_body(c_ref, w_ref, o_ref):
    o_ref[...] = jnp.dot(c_ref[...], w_ref[...], preferred_element_type=jnp.float32)


def _tc_matmul(coeff, weight):
    return pl.pallas_call(
        _mm_body,
        grid=(B // _MBLK,),
        in_specs=[
            pl.BlockSpec((_MBLK, VOCAB), lambda i: (i, 0)),
            pl.BlockSpec((VOCAB, DIM), lambda i: (0, 0)),
        ],
        out_specs=pl.BlockSpec((_MBLK, DIM), lambda i: (i, 0)),
        out_shape=jax.ShapeDtypeStruct((B, DIM), jnp.float32),
    )(coeff, weight)


def kernel(indices, per_sample_weights, weight):
    idxp = jnp.pad(indices.astype(jnp.int32), ((0, 0), (0, LP - L)))
    pswp = jnp.pad(per_sample_weights, ((0, 0), (0, LP - L)))
    coeff = _sc_coeff(idxp.reshape(-1), pswp.reshape(-1))
    return _tc_matmul(coeff.reshape(B, VOCAB), weight)


# speedup vs baseline: 40.6982x; 1.8030x over previous
import functools

import jax
import jax.numpy as jnp
from jax import lax
from jax.experimental import pallas as pl
from jax.experimental.pallas import tpu as pltpu
from jax.experimental.pallas import tpu_sc as plsc

B = 16384
L = 50
VOCAB = 256
DIM = 256

_NC, _NS = 2, 16
_NW = _NC * _NS
_RPW = B // _NW
_SUB = 128
_NSUB = _RPW // _SUB
_GRP = _SUB // 16


def _sc_body(idx_hbm, psw_hbm, coeff_hbm, idx_v, psw_v, acc_v):
    wid = lax.axis_index("s") * _NC + lax.axis_index("c")
    base = wid * _RPW
    lane = lax.iota(jnp.int32, 16)
    gath_base = lane * L
    row_base = lane * VOCAB
    zeros16 = jnp.zeros((16,), jnp.float32)

    for sub in range(_NSUB):
        r0 = base + sub * _SUB
        pltpu.sync_copy(idx_hbm.at[pl.ds(r0 * L, _SUB * L)], idx_v)
        pltpu.sync_copy(psw_hbm.at[pl.ds(r0 * L, _SUB * L)], psw_v)

        @plsc.parallel_loop(0, _SUB * VOCAB, step=128)
        def _zero(i):
            for k in range(8):
                acc_v[pl.ds(i + k * 16, 16)] = zeros16

        @plsc.parallel_loop(0, L, unroll=2)
        def _scatter(l):
            for g in range(_GRP):
                gidx = gath_base + (g * 16 * L) + l
                ivals = plsc.load_gather(idx_v, [gidx])
                pvals = plsc.load_gather(psw_v, [gidx])
                dest = (row_base + g * 16 * VOCAB) + ivals
                plsc.addupdate_scatter(acc_v, [dest], pvals)

        pltpu.sync_copy(acc_v, coeff_hbm.at[pl.ds(r0 * VOCAB, _SUB * VOCAB)])


_sc_coeff = functools.partial(
    pl.kernel,
    out_type=jax.ShapeDtypeStruct((B * VOCAB,), jnp.float32),
    mesh=plsc.VectorSubcoreMesh(
        core_axis_name="c", subcore_axis_name="s", num_cores=_NC, num_subcores=_NS
    ),
    scratch_types=[
        pltpu.VMEM((_SUB * L,), jnp.int32),
        pltpu.VMEM((_SUB * L,), jnp.float32),
        pltpu.VMEM((_SUB * VOCAB,), jnp.float32),
    ],
    compiler_params=pltpu.CompilerParams(needs_layout_passes=False),
)(_sc_body)


_MBLK = 2048


def _mm_body(c_ref, w_ref, o_ref):
    o_ref[...] = jnp.dot(c_ref[...], w_ref[...], preferred_element_type=jnp.float32)


def _tc_matmul(coeff, weight):
    return pl.pallas_call(
        _mm_body,
        grid=(B // _MBLK,),
        in_specs=[
            pl.BlockSpec((_MBLK, VOCAB), lambda i: (i, 0)),
            pl.BlockSpec((VOCAB, DIM), lambda i: (0, 0)),
        ],
        out_specs=pl.BlockSpec((_MBLK, DIM), lambda i: (i, 0)),
        out_shape=jax.ShapeDtypeStruct((B, DIM), jnp.float32),
    )(coeff, weight)


def kernel(indices, per_sample_weights, weight):
    idx32 = indices.astype(jnp.int32).reshape(-1)
    coeff = _sc_coeff(idx32, per_sample_weights.reshape(-1))
    return _tc_matmul(coeff.reshape(B, VOCAB), weight)


# speedup vs baseline: 49.5821x; 1.2183x over previous
import functools

import jax
import jax.numpy as jnp
from jax import lax
from jax.experimental import pallas as pl
from jax.experimental.pallas import tpu as pltpu
from jax.experimental.pallas import tpu_sc as plsc

B = 16384
L = 50
VOCAB = 256
DIM = 256

_NC, _NS = 2, 16
_NW = _NC * _NS
_RPW = B // _NW
_SUB = 128
_NSUB = _RPW // _SUB
_GRP = _SUB // 16


def _sc_body(idx_hbm, psw_hbm, coeff_hbm, idx_v, psw_v, acc_v):
    wid = lax.axis_index("s") * _NC + lax.axis_index("c")
    base = wid * _RPW
    lane = lax.iota(jnp.int32, 16)
    gath_base = lane * L
    zeros16 = jnp.zeros((16,), jnp.float32)

    for sub in range(_NSUB):
        r0 = base + sub * _SUB
        pltpu.sync_copy(idx_hbm.at[pl.ds(r0 * L, _SUB * L)], idx_v)
        pltpu.sync_copy(psw_hbm.at[pl.ds(r0 * L, _SUB * L)], psw_v)

        @plsc.parallel_loop(0, _SUB)
        def _zero(r):
            for k in range(VOCAB // 16):
                acc_v[r, pl.ds(k * 16, 16)] = zeros16

        @plsc.parallel_loop(0, L, unroll=2)
        def _scatter(l):
            for g in range(_GRP):
                gidx = gath_base + (g * 16 * L) + l
                ivals = plsc.load_gather(idx_v, [gidx])
                pvals = plsc.load_gather(psw_v, [gidx])
                rows = lane + g * 16
                plsc.addupdate_scatter(acc_v, [rows, ivals], pvals)

        pltpu.sync_copy(acc_v, coeff_hbm.at[pl.ds(r0, _SUB)])


_sc_coeff = functools.partial(
    pl.kernel,
    out_type=jax.ShapeDtypeStruct((B, VOCAB), jnp.float32),
    mesh=plsc.VectorSubcoreMesh(
        core_axis_name="c", subcore_axis_name="s", num_cores=_NC, num_subcores=_NS
    ),
    scratch_types=[
        pltpu.VMEM((_SUB * L,), jnp.int32),
        pltpu.VMEM((_SUB * L,), jnp.float32),
        pltpu.VMEM((_SUB, VOCAB), jnp.float32),
    ],
    compiler_params=pltpu.CompilerParams(needs_layout_passes=False),
)(_sc_body)


_MBLK = 2048


def _mm_body(c_ref, w_ref, o_ref):
    o_ref[...] = jnp.dot(c_ref[...], w_ref[...], preferred_element_type=jnp.float32)


def _tc_matmul(coeff, weight):
    return pl.pallas_call(
        _mm_body,
        grid=(B // _MBLK,),
        in_specs=[
            pl.BlockSpec((_MBLK, VOCAB), lambda i: (i, 0)),
            pl.BlockSpec((VOCAB, DIM), lambda i: (0, 0)),
        ],
        out_specs=pl.BlockSpec((_MBLK, DIM), lambda i: (i, 0)),
        out_shape=jax.ShapeDtypeStruct((B, DIM), jnp.float32),
    )(coeff, weight)


def kernel(indices, per_sample_weights, weight):
    idx32 = indices.astype(jnp.int32).reshape(-1)
    coeff = _sc_coeff(idx32, per_sample_weights.reshape(-1))
    return _tc_matmul(coeff, weight)
